# baseline (speedup 1.0000x reference)
import functools

import jax
import jax.numpy as jnp
from jax import lax
from jax.experimental import pallas as pl
from jax.experimental.pallas import tpu as pltpu
from jax.experimental.pallas import tpu_sc as plsc

N = 10000
E = 320000
D = 128
NC = 2
NS = 16
NW = NC * NS
EPT = E // NW
CH = 128
EPTP = 10240
NCHUNK = EPTP // CH
EPAD = NW * EPTP - E
NP = 10240
RPT = NP // NS

_MESH = plsc.VectorSubcoreMesh(core_axis_name="c", subcore_axis_name="s")


def _sc_agg_body(compute_deg, *refs):
    if compute_deg:
        (h_hbm, pk_hbm, z2_hbm, z1_hbm, ones_hbm,
         acc_out, deg_out,
         pk_v, src_c0, dst_c0, src_c1, dst_c1, rows0, rows1, ones_v,
         acc_sh, deg_sh, sem0, sem1) = refs
    else:
        (h_hbm, pk_hbm, z2_hbm,
         acc_out,
         pk_v, src_c0, dst_c0, src_c1, dst_c1, rows0, rows1,
         acc_sh, sem0, sem1) = refs

    c = lax.axis_index("c")
    s = lax.axis_index("s")
    wid = c * NS + s

    pltpu.sync_copy(pk_hbm.at[wid], pk_v)

    def unpack(k, src_c, dst_c):
        for j in range(CH // 16):
            p = pk_v[k, pl.ds(j * 16, 16)]
            src_c[pl.ds(j * 16, 16)] = jnp.bitwise_and(p, 0xFFFF)
            dst_c[pl.ds(j * 16, 16)] = jnp.right_shift(p, 16)

    def scatter(rows, dst_c):
        pltpu.sync_copy(rows, acc_sh.at[dst_c], add=True)
        if compute_deg:
            pltpu.sync_copy(ones_v, deg_sh.at[dst_c], add=True)

    unpack(0, src_c0, dst_c0)
    unpack(1, src_c1, dst_c1)
    pltpu.async_copy(h_hbm.at[src_c0], rows0, sem0)
    pltpu.async_copy(h_hbm.at[src_c1], rows1, sem1)

    pltpu.sync_copy(z2_hbm, acc_sh.at[pl.ds(s * RPT, RPT)])
    if compute_deg:
        pltpu.sync_copy(z1_hbm, deg_sh.at[pl.ds(s * RPT, RPT)])
        pltpu.sync_copy(ones_hbm, ones_v)
    plsc.subcore_barrier()

    def step(k, src_c, dst_c, rows, sem, prefetch):
        pltpu.make_async_copy(h_hbm.at[src_c], rows, sem).wait()
        scatter(rows, dst_c)
        if prefetch:
            unpack(k + 2, src_c, dst_c)
            pltpu.async_copy(h_hbm.at[src_c], rows, sem)

    def body(i, carry):
        k = 2 * i
        step(k, src_c0, dst_c0, rows0, sem0, True)
        step(k + 1, src_c1, dst_c1, rows1, sem1, True)
        return carry

    if NCHUNK % 2 == 0:
        lax.fori_loop(0, NCHUNK // 2 - 1, body, 0)
        step(NCHUNK - 2, src_c0, dst_c0, rows0, sem0, False)
        step(NCHUNK - 1, src_c1, dst_c1, rows1, sem1, False)
    else:
        lax.fori_loop(0, (NCHUNK - 3) // 2, body, 0)
        step(NCHUNK - 3, src_c0, dst_c0, rows0, sem0, True)
        step(NCHUNK - 2, src_c1, dst_c1, rows1, sem1, False)
        step(NCHUNK - 1, src_c0, dst_c0, rows0, sem0, False)
    plsc.subcore_barrier()

    pltpu.sync_copy(acc_sh.at[pl.ds(s * RPT, RPT)],
                    acc_out.at[c, pl.ds(s * RPT, RPT)])
    if compute_deg:
        pltpu.sync_copy(deg_sh.at[pl.ds(s * RPT, RPT)],
                        deg_out.at[c, pl.ds(s * RPT, RPT)])


_sc_agg_deg = pl.kernel(
    functools.partial(_sc_agg_body, True),
    out_type=(
        jax.ShapeDtypeStruct((NC, NP, D), jnp.float32),
        jax.ShapeDtypeStruct((NC, NP), jnp.float32),
    ),
    mesh=_MESH,
    scratch_types=[
        pltpu.VMEM((NCHUNK, CH), jnp.int32),
        pltpu.VMEM((CH,), jnp.int32),
        pltpu.VMEM((CH,), jnp.int32),
        pltpu.VMEM((CH,), jnp.int32),
        pltpu.VMEM((CH,), jnp.int32),
        pltpu.VMEM((CH, D), jnp.float32),
        pltpu.VMEM((CH, D), jnp.float32),
        pltpu.VMEM((CH,), jnp.float32),
        pltpu.VMEM_SHARED((NP, D), jnp.float32),
        pltpu.VMEM_SHARED((NP,), jnp.float32),
        pltpu.SemaphoreType.DMA,
        pltpu.SemaphoreType.DMA,
    ],
    name="sc_agg_deg",
)

_sc_agg = pl.kernel(
    functools.partial(_sc_agg_body, False),
    out_type=jax.ShapeDtypeStruct((NC, NP, D), jnp.float32),
    mesh=_MESH,
    scratch_types=[
        pltpu.VMEM((NCHUNK, CH), jnp.int32),
        pltpu.VMEM((CH,), jnp.int32),
        pltpu.VMEM((CH,), jnp.int32),
        pltpu.VMEM((CH,), jnp.int32),
        pltpu.VMEM((CH,), jnp.int32),
        pltpu.VMEM((CH, D), jnp.float32),
        pltpu.VMEM((CH, D), jnp.float32),
        pltpu.VMEM_SHARED((NP, D), jnp.float32),
        pltpu.SemaphoreType.DMA,
        pltpu.SemaphoreType.DMA,
    ],
    name="sc_agg",
)

EB = E // 128


def _pack_body(s_ref, d_ref, out_ref):
    out_ref[...] = s_ref[0] | (d_ref[0] << 16)


_tc_pack = pl.pallas_call(
    _pack_body,
    grid=(1,),
    in_specs=[pl.BlockSpec((1, EB, 128), lambda i: (0, 0, 0)),
              pl.BlockSpec((1, EB, 128), lambda i: (1, 0, 0))],
    out_specs=pl.BlockSpec((EB, 128), lambda i: (0, 0)),
    out_shape=jax.ShapeDtypeStruct((EB, 128), jnp.int32),
)


BN = 1000


def _tc_lin_body(x_ref, w_ref, b_ref, out_ref):
    out_ref[...] = (jnp.dot(x_ref[...], w_ref[...],
                            preferred_element_type=jnp.float32)
                    + b_ref[...])


def _tc_comb1_body(lin_ref, a0_ref, a1_ref, deg_ref, wn_ref, h_out):
    n = (a0_ref[0] + a1_ref[0]) / deg_ref[...]
    h = lin_ref[...] + jnp.dot(n, wn_ref[...],
                               preferred_element_type=jnp.float32)
    h_out[...] = jnp.maximum(h, 0.0)


def _tc_comb2_body(lin_ref, a0_ref, a1_ref, deg_ref, wn_ref, out_ref):
    n = (a0_ref[0] + a1_ref[0]) / deg_ref[...]
    out_ref[...] = lin_ref[...] + jnp.dot(n, wn_ref[...],
                                          preferred_element_type=jnp.float32)


def _row_spec():
    return pl.BlockSpec((BN, D), lambda i: (i, 0))


def _acc_spec(c):
    return pl.BlockSpec((1, BN, D), lambda i, c=c: (c, i, 0))


def _col_spec():
    return pl.BlockSpec((BN, 1), lambda i: (i, 0))


def _full_spec():
    return pl.BlockSpec((D, D), lambda i: (0, 0))


def _bias_spec():
    return pl.BlockSpec((1, D), lambda i: (0, 0))


_tc_lin = pl.pallas_call(
    _tc_lin_body,
    grid=(N // BN,),
    in_specs=[_row_spec(), _full_spec(), _bias_spec()],
    out_specs=_row_spec(),
    out_shape=jax.ShapeDtypeStruct((N, D), jnp.float32),
)

_tc_comb1 = pl.pallas_call(
    _tc_comb1_body,
    grid=(N // BN,),
    in_specs=[_row_spec(), _acc_spec(0), _acc_spec(1), _col_spec(),
              _full_spec()],
    out_specs=_row_spec(),
    out_shape=jax.ShapeDtypeStruct((N, D), jnp.float32),
)

_tc_comb2 = pl.pallas_call(
    _tc_comb2_body,
    grid=(N // BN,),
    in_specs=[_row_spec(), _acc_spec(0), _acc_spec(1), _col_spec(),
              _full_spec()],
    out_specs=_row_spec(),
    out_shape=jax.ShapeDtypeStruct((N, D), jnp.float32),
)


def kernel(x, edge_index, W_self1, W_neigh1, b1, W_self2, W_neigh2, b2,
           W_fc, b_fc):
    ei = edge_index.astype(jnp.int32).reshape(2, EB, 128)
    pad = jnp.full((EPAD,), (NP - 1) << 16, jnp.int32)
    pk = jnp.concatenate([_tc_pack(ei, ei).reshape(E), pad])
    pk = pk.reshape(NW, NCHUNK, CH)
    z2 = jnp.zeros((RPT, D), jnp.float32)
    z1 = jnp.zeros((RPT,), jnp.float32)
    ones = jnp.ones((CH,), jnp.float32)

    lin1 = _tc_lin(x, W_self1.T, b1.reshape(1, D))
    acc1, degp = _sc_agg_deg(x, pk, z2, z1, ones)
    deg = jnp.maximum(degp[0, :N] + degp[1, :N], 1.0).reshape(N, 1)
    h1 = _tc_comb1(lin1, acc1, acc1, deg, W_neigh1.T)
    lin2 = _tc_lin(h1, (W_fc @ W_self2).T,
                   (b2 @ W_fc.T + b_fc).reshape(1, D))
    acc2 = _sc_agg(h1, pk, z2)
    out = _tc_comb2(lin2, acc2, acc2, deg, (W_fc @ W_neigh2).T)
    return out


# speedup vs baseline: 3.4565x; 3.4565x over previous
import functools

import jax
import jax.numpy as jnp
from jax import lax
from jax.experimental import pallas as pl
from jax.experimental.pallas import tpu as pltpu
from jax.experimental.pallas import tpu_sc as plsc

N = 10000
E = 320000
D = 128
NC = 2
NS = 16
NW = NC * NS
EPT = E // NW
CH = 80
NCHUNK = EPT // CH
NP = 10240
RPT = NP // NS

_MESH = plsc.VectorSubcoreMesh(core_axis_name="c", subcore_axis_name="s")


def _sc_agg_body(compute_deg, *refs):
    if compute_deg:
        (h_hbm, pk_hbm, z2_hbm, z1_hbm, ones_hbm,
         acc_out, deg_out,
         pk_v, src_c0, dst_c0, src_c1, dst_c1, rows0, rows1, ones_v,
         acc_sh, deg_sh, sem0, sem1) = refs
    else:
        (h_hbm, pk_hbm, z2_hbm,
         acc_out,
         pk_v, src_c0, dst_c0, src_c1, dst_c1, rows0, rows1,
         acc_sh, sem0, sem1) = refs

    c = lax.axis_index("c")
    s = lax.axis_index("s")
    wid = c * NS + s

    pltpu.sync_copy(pk_hbm.at[wid], pk_v)

    def unpack(k, src_c, dst_c):
        for j in range(CH // 16):
            p = pk_v[k, pl.ds(j * 16, 16)]
            src_c[pl.ds(j * 16, 16)] = jnp.bitwise_and(p, 0xFFFF)
            dst_c[pl.ds(j * 16, 16)] = jnp.right_shift(p, 16)

    def scatter(rows, dst_c):
        pltpu.sync_copy(rows, acc_sh.at[dst_c], add=True)
        if compute_deg:
            pltpu.sync_copy(ones_v, deg_sh.at[dst_c], add=True)

    unpack(0, src_c0, dst_c0)
    unpack(1, src_c1, dst_c1)
    pltpu.async_copy(h_hbm.at[src_c0], rows0, sem0)
    pltpu.async_copy(h_hbm.at[src_c1], rows1, sem1)

    pltpu.sync_copy(z2_hbm, acc_sh.at[pl.ds(s * RPT, RPT)])
    if compute_deg:
        pltpu.sync_copy(z1_hbm, deg_sh.at[pl.ds(s * RPT, RPT)])
        pltpu.sync_copy(ones_hbm, ones_v)
    plsc.subcore_barrier()

    def step(k, src_c, dst_c, rows, sem, prefetch):
        pltpu.make_async_copy(h_hbm.at[src_c], rows, sem).wait()
        scatter(rows, dst_c)
        if prefetch:
            unpack(k + 2, src_c, dst_c)
            pltpu.async_copy(h_hbm.at[src_c], rows, sem)

    def body(i, carry):
        k = 2 * i
        step(k, src_c0, dst_c0, rows0, sem0, True)
        step(k + 1, src_c1, dst_c1, rows1, sem1, True)
        return carry

    if NCHUNK % 2 == 0:
        lax.fori_loop(0, NCHUNK // 2 - 1, body, 0)
        step(NCHUNK - 2, src_c0, dst_c0, rows0, sem0, False)
        step(NCHUNK - 1, src_c1, dst_c1, rows1, sem1, False)
    else:
        lax.fori_loop(0, (NCHUNK - 3) // 2, body, 0)
        step(NCHUNK - 3, src_c0, dst_c0, rows0, sem0, True)
        step(NCHUNK - 2, src_c1, dst_c1, rows1, sem1, False)
        step(NCHUNK - 1, src_c0, dst_c0, rows0, sem0, False)
    plsc.subcore_barrier()

    pltpu.sync_copy(acc_sh.at[pl.ds(s * RPT, RPT)],
                    acc_out.at[c, pl.ds(s * RPT, RPT)])
    if compute_deg:
        pltpu.sync_copy(deg_sh.at[pl.ds(s * RPT, RPT)],
                        deg_out.at[c, pl.ds(s * RPT, RPT)])


_sc_agg_deg = pl.kernel(
    functools.partial(_sc_agg_body, True),
    out_type=(
        jax.ShapeDtypeStruct((NC, NP, D), jnp.float32),
        jax.ShapeDtypeStruct((NC, NP), jnp.float32),
    ),
    mesh=_MESH,
    scratch_types=[
        pltpu.VMEM((NCHUNK, CH), jnp.int32),
        pltpu.VMEM((CH,), jnp.int32),
        pltpu.VMEM((CH,), jnp.int32),
        pltpu.VMEM((CH,), jnp.int32),
        pltpu.VMEM((CH,), jnp.int32),
        pltpu.VMEM((CH, D), jnp.float32),
        pltpu.VMEM((CH, D), jnp.float32),
        pltpu.VMEM((CH,), jnp.float32),
        pltpu.VMEM_SHARED((NP, D), jnp.float32),
        pltpu.VMEM_SHARED((NP,), jnp.float32),
        pltpu.SemaphoreType.DMA,
        pltpu.SemaphoreType.DMA,
    ],
    name="sc_agg_deg",
)

_sc_agg = pl.kernel(
    functools.partial(_sc_agg_body, False),
    out_type=jax.ShapeDtypeStruct((NC, NP, D), jnp.float32),
    mesh=_MESH,
    scratch_types=[
        pltpu.VMEM((NCHUNK, CH), jnp.int32),
        pltpu.VMEM((CH,), jnp.int32),
        pltpu.VMEM((CH,), jnp.int32),
        pltpu.VMEM((CH,), jnp.int32),
        pltpu.VMEM((CH,), jnp.int32),
        pltpu.VMEM((CH, D), jnp.float32),
        pltpu.VMEM((CH, D), jnp.float32),
        pltpu.VMEM_SHARED((NP, D), jnp.float32),
        pltpu.SemaphoreType.DMA,
        pltpu.SemaphoreType.DMA,
    ],
    name="sc_agg",
)

EB = E // 128


def _pack_body(s_ref, d_ref, out_ref):
    out_ref[...] = s_ref[0] | (d_ref[0] << 16)


_tc_pack = pl.pallas_call(
    _pack_body,
    grid=(1,),
    in_specs=[pl.BlockSpec((1, EB, 128), lambda i: (0, 0, 0)),
              pl.BlockSpec((1, EB, 128), lambda i: (1, 0, 0))],
    out_specs=pl.BlockSpec((EB, 128), lambda i: (0, 0)),
    out_shape=jax.ShapeDtypeStruct((EB, 128), jnp.int32),
)


BN = 1000


def _tc_lin_body(x_ref, w_ref, b_ref, out_ref):
    out_ref[...] = (jnp.dot(x_ref[...], w_ref[...],
                            preferred_element_type=jnp.float32)
                    + b_ref[...])


def _tc_comb1_body(lin_ref, a0_ref, a1_ref, deg_ref, wn_ref, h_out):
    n = (a0_ref[0] + a1_ref[0]) / deg_ref[...]
    h = lin_ref[...] + jnp.dot(n, wn_ref[...],
                               preferred_element_type=jnp.float32)
    h_out[...] = jnp.maximum(h, 0.0)


def _tc_comb2_body(lin_ref, a0_ref, a1_ref, deg_ref, wn_ref, out_ref):
    n = (a0_ref[0] + a1_ref[0]) / deg_ref[...]
    out_ref[...] = lin_ref[...] + jnp.dot(n, wn_ref[...],
                                          preferred_element_type=jnp.float32)


def _row_spec():
    return pl.BlockSpec((BN, D), lambda i: (i, 0))


def _acc_spec(c):
    return pl.BlockSpec((1, BN, D), lambda i, c=c: (c, i, 0))


def _col_spec():
    return pl.BlockSpec((BN, 1), lambda i: (i, 0))


def _full_spec():
    return pl.BlockSpec((D, D), lambda i: (0, 0))


def _bias_spec():
    return pl.BlockSpec((1, D), lambda i: (0, 0))


_tc_lin = pl.pallas_call(
    _tc_lin_body,
    grid=(N // BN,),
    in_specs=[_row_spec(), _full_spec(), _bias_spec()],
    out_specs=_row_spec(),
    out_shape=jax.ShapeDtypeStruct((N, D), jnp.float32),
)

_tc_comb1 = pl.pallas_call(
    _tc_comb1_body,
    grid=(N // BN,),
    in_specs=[_row_spec(), _acc_spec(0), _acc_spec(1), _col_spec(),
              _full_spec()],
    out_specs=_row_spec(),
    out_shape=jax.ShapeDtypeStruct((N, D), jnp.float32),
)

_tc_comb2 = pl.pallas_call(
    _tc_comb2_body,
    grid=(N // BN,),
    in_specs=[_row_spec(), _acc_spec(0), _acc_spec(1), _col_spec(),
              _full_spec()],
    out_specs=_row_spec(),
    out_shape=jax.ShapeDtypeStruct((N, D), jnp.float32),
)


def kernel(x, edge_index, W_self1, W_neigh1, b1, W_self2, W_neigh2, b2,
           W_fc, b_fc):
    ei = edge_index.astype(jnp.int32).reshape(2, EB, 128)
    pk = _tc_pack(ei, ei).reshape(NW, NCHUNK, CH)
    z2 = jnp.zeros((RPT, D), jnp.float32)
    z1 = jnp.zeros((RPT,), jnp.float32)
    ones = jnp.ones((CH,), jnp.float32)

    lin1 = _tc_lin(x, W_self1.T, b1.reshape(1, D))
    acc1, degp = _sc_agg_deg(x, pk, z2, z1, ones)
    deg = jnp.maximum(degp[0, :N] + degp[1, :N], 1.0).reshape(N, 1)
    h1 = _tc_comb1(lin1, acc1, acc1, deg, W_neigh1.T)
    lin2 = _tc_lin(h1, (W_fc @ W_self2).T,
                   (b2 @ W_fc.T + b_fc).reshape(1, D))
    acc2 = _sc_agg(h1, pk, z2)
    out = _tc_comb2(lin2, acc2, acc2, deg, (W_fc @ W_neigh2).T)
    return out


# speedup vs baseline: 3.7330x; 1.0800x over previous
import functools

import jax
import jax.numpy as jnp
from jax import lax
from jax.experimental import pallas as pl
from jax.experimental.pallas import tpu as pltpu
from jax.experimental.pallas import tpu_sc as plsc

N = 10000
E = 320000
D = 128
NC = 2
NS = 16
NW = NC * NS
EPT = E // NW
CH = 128
EPTP = 10240
NCHUNK = EPTP // CH
EPAD = NW * EPTP - E
NP = 10240
RPT = NP // NS

_MESH = plsc.VectorSubcoreMesh(core_axis_name="c", subcore_axis_name="s")


def _sc_agg_body(compute_deg, *refs):
    if compute_deg:
        (h_hbm, pk_hbm, z2_hbm, z1_hbm, ones_hbm,
         acc_out, deg_out,
         pk_v, src_c0, dst_c0, src_c1, dst_c1, rows0, rows1, ones_v,
         acc_sh, deg_sh, sem0, sem1) = refs
    else:
        (h_hbm, pk_hbm, z2_hbm,
         acc_out,
         pk_v, src_c0, dst_c0, src_c1, dst_c1, rows0, rows1,
         acc_sh, sem0, sem1) = refs

    c = lax.axis_index("c")
    s = lax.axis_index("s")
    wid = c * NS + s

    pltpu.sync_copy(pk_hbm.at[wid], pk_v)

    def unpack(k, src_c, dst_c):
        for j in range(CH // 16):
            p = pk_v[k, pl.ds(j * 16, 16)]
            src_c[pl.ds(j * 16, 16)] = jnp.bitwise_and(p, 0xFFFF)
            dst_c[pl.ds(j * 16, 16)] = jnp.right_shift(p, 16)

    def scatter(rows, dst_c):
        pltpu.sync_copy(rows, acc_sh.at[dst_c], add=True)
        if compute_deg:
            pltpu.sync_copy(ones_v, deg_sh.at[dst_c], add=True)

    unpack(0, src_c0, dst_c0)
    unpack(1, src_c1, dst_c1)
    pltpu.async_copy(h_hbm.at[src_c0], rows0, sem0)
    pltpu.async_copy(h_hbm.at[src_c1], rows1, sem1)

    pltpu.sync_copy(z2_hbm, acc_sh.at[pl.ds(s * RPT, RPT)])
    if compute_deg:
        pltpu.sync_copy(z1_hbm, deg_sh.at[pl.ds(s * RPT, RPT)])
        pltpu.sync_copy(ones_hbm, ones_v)
    plsc.subcore_barrier()

    def step(k, src_c, dst_c, rows, sem, prefetch):
        pltpu.make_async_copy(h_hbm.at[src_c], rows, sem).wait()
        scatter(rows, dst_c)
        if prefetch:
            unpack(k + 2, src_c, dst_c)
            pltpu.async_copy(h_hbm.at[src_c], rows, sem)

    def body(i, carry):
        k = 2 * i
        step(k, src_c0, dst_c0, rows0, sem0, True)
        step(k + 1, src_c1, dst_c1, rows1, sem1, True)
        return carry

    if NCHUNK % 2 == 0:
        lax.fori_loop(0, NCHUNK // 2 - 1, body, 0)
        step(NCHUNK - 2, src_c0, dst_c0, rows0, sem0, False)
        step(NCHUNK - 1, src_c1, dst_c1, rows1, sem1, False)
    else:
        lax.fori_loop(0, (NCHUNK - 3) // 2, body, 0)
        step(NCHUNK - 3, src_c0, dst_c0, rows0, sem0, True)
        step(NCHUNK - 2, src_c1, dst_c1, rows1, sem1, False)
        step(NCHUNK - 1, src_c0, dst_c0, rows0, sem0, False)
    plsc.subcore_barrier()

    pltpu.sync_copy(acc_sh.at[pl.ds(s * RPT, RPT)],
                    acc_out.at[c, pl.ds(s * RPT, RPT)])
    if compute_deg:
        pltpu.sync_copy(deg_sh.at[pl.ds(s * RPT, RPT)],
                        deg_out.at[c, pl.ds(s * RPT, RPT)])


_sc_agg_deg = pl.kernel(
    functools.partial(_sc_agg_body, True),
    out_type=(
        jax.ShapeDtypeStruct((NC, NP, D), jnp.float32),
        jax.ShapeDtypeStruct((NC, NP), jnp.float32),
    ),
    mesh=_MESH,
    scratch_types=[
        pltpu.VMEM((NCHUNK, CH), jnp.int32),
        pltpu.VMEM((CH,), jnp.int32),
        pltpu.VMEM((CH,), jnp.int32),
        pltpu.VMEM((CH,), jnp.int32),
        pltpu.VMEM((CH,), jnp.int32),
        pltpu.VMEM((CH, D), jnp.float32),
        pltpu.VMEM((CH, D), jnp.float32),
        pltpu.VMEM((CH,), jnp.float32),
        pltpu.VMEM_SHARED((NP, D), jnp.float32),
        pltpu.VMEM_SHARED((NP,), jnp.float32),
        pltpu.SemaphoreType.DMA,
        pltpu.SemaphoreType.DMA,
    ],
    name="sc_agg_deg",
)

_sc_agg = pl.kernel(
    functools.partial(_sc_agg_body, False),
    out_type=jax.ShapeDtypeStruct((NC, NP, D), jnp.float32),
    mesh=_MESH,
    scratch_types=[
        pltpu.VMEM((NCHUNK, CH), jnp.int32),
        pltpu.VMEM((CH,), jnp.int32),
        pltpu.VMEM((CH,), jnp.int32),
        pltpu.VMEM((CH,), jnp.int32),
        pltpu.VMEM((CH,), jnp.int32),
        pltpu.VMEM((CH, D), jnp.float32),
        pltpu.VMEM((CH, D), jnp.float32),
        pltpu.VMEM_SHARED((NP, D), jnp.float32),
        pltpu.SemaphoreType.DMA,
        pltpu.SemaphoreType.DMA,
    ],
    name="sc_agg",
)

EB = E // 128


def _pack_body(s_ref, d_ref, out_ref):
    out_ref[...] = s_ref[0] | (d_ref[0] << 16)


_tc_pack = pl.pallas_call(
    _pack_body,
    grid=(1,),
    in_specs=[pl.BlockSpec((1, EB, 128), lambda i: (0, 0, 0)),
              pl.BlockSpec((1, EB, 128), lambda i: (1, 0, 0))],
    out_specs=pl.BlockSpec((EB, 128), lambda i: (0, 0)),
    out_shape=jax.ShapeDtypeStruct((EB, 128), jnp.int32),
)


BN = 1000


def _tc_lin_body(x_ref, w_ref, b_ref, out_ref):
    out_ref[...] = (jnp.dot(x_ref[...], w_ref[...],
                            preferred_element_type=jnp.float32)
                    + b_ref[...])


def _tc_comb1_body(lin_ref, a0_ref, a1_ref, deg_ref, wn_ref, h_out):
    n = (a0_ref[0] + a1_ref[0]) / deg_ref[...]
    h = lin_ref[...] + jnp.dot(n, wn_ref[...],
                               preferred_element_type=jnp.float32)
    h_out[...] = jnp.maximum(h, 0.0)


def _tc_comb2_body(lin_ref, a0_ref, a1_ref, deg_ref, wn_ref, out_ref):
    n = (a0_ref[0] + a1_ref[0]) / deg_ref[...]
    out_ref[...] = lin_ref[...] + jnp.dot(n, wn_ref[...],
                                          preferred_element_type=jnp.float32)


def _row_spec():
    return pl.BlockSpec((BN, D), lambda i: (i, 0))


def _acc_spec(c):
    return pl.BlockSpec((1, BN, D), lambda i, c=c: (c, i, 0))


def _col_spec():
    return pl.BlockSpec((BN, 1), lambda i: (i, 0))


def _full_spec():
    return pl.BlockSpec((D, D), lambda i: (0, 0))


def _bias_spec():
    return pl.BlockSpec((1, D), lambda i: (0, 0))


_tc_lin = pl.pallas_call(
    _tc_lin_body,
    grid=(N // BN,),
    in_specs=[_row_spec(), _full_spec(), _bias_spec()],
    out_specs=_row_spec(),
    out_shape=jax.ShapeDtypeStruct((N, D), jnp.float32),
)

_tc_comb1 = pl.pallas_call(
    _tc_comb1_body,
    grid=(N // BN,),
    in_specs=[_row_spec(), _acc_spec(0), _acc_spec(1), _col_spec(),
              _full_spec()],
    out_specs=_row_spec(),
    out_shape=jax.ShapeDtypeStruct((N, D), jnp.float32),
)

_tc_comb2 = pl.pallas_call(
    _tc_comb2_body,
    grid=(N // BN,),
    in_specs=[_row_spec(), _acc_spec(0), _acc_spec(1), _col_spec(),
              _full_spec()],
    out_specs=_row_spec(),
    out_shape=jax.ShapeDtypeStruct((N, D), jnp.float32),
)


def kernel(x, edge_index, W_self1, W_neigh1, b1, W_self2, W_neigh2, b2,
           W_fc, b_fc):
    ei = edge_index.astype(jnp.int32).reshape(2, EB, 128)
    ar = jnp.arange(EPAD, dtype=jnp.int32)
    pad = (ar % N) | ((N + ar % (NP - N)) << 16)
    pk = jnp.concatenate([_tc_pack(ei, ei).reshape(E), pad])
    pk = pk.reshape(NW, NCHUNK, CH)
    z2 = jnp.zeros((RPT, D), jnp.float32)
    z1 = jnp.zeros((RPT,), jnp.float32)
    ones = jnp.ones((CH,), jnp.float32)

    lin1 = _tc_lin(x, W_self1.T, b1.reshape(1, D))
    acc1, degp = _sc_agg_deg(x, pk, z2, z1, ones)
    deg = jnp.maximum(degp[0, :N] + degp[1, :N], 1.0).reshape(N, 1)
    h1 = _tc_comb1(lin1, acc1, acc1, deg, W_neigh1.T)
    lin2 = _tc_lin(h1, (W_fc @ W_self2).T,
                   (b2 @ W_fc.T + b_fc).reshape(1, D))
    acc2 = _sc_agg(h1, pk, z2)
    out = _tc_comb2(lin2, acc2, acc2, deg, (W_fc @ W_neigh2).T)
    return out


# speedup vs baseline: 3.7664x; 1.0090x over previous
import functools

import jax
import jax.numpy as jnp
from jax import lax
from jax.experimental import pallas as pl
from jax.experimental.pallas import tpu as pltpu
from jax.experimental.pallas import tpu_sc as plsc

N = 10000
E = 320000
D = 128
NC = 2
NS = 16
NW = NC * NS
EPT = E // NW
CH = 128
EPTP = 10240
NCHUNK = EPTP // CH
EPAD = NW * EPTP - E
NP = 10240
RPT = NP // NS

_MESH = plsc.VectorSubcoreMesh(core_axis_name="c", subcore_axis_name="s")


def _sc_agg_body(compute_deg, *refs):
    if compute_deg:
        (h_hbm, pk_hbm, z2_hbm, z1_hbm, ones_hbm,
         acc_out, deg_out,
         pk_v, src_c0, dst_c0, src_c1, dst_c1, rows0, rows1, ones_v,
         acc_sh, deg_sh, sem0, sem1) = refs
    else:
        (h_hbm, pk_hbm, z2_hbm,
         acc_out,
         pk_v, src_c0, dst_c0, src_c1, dst_c1, rows0, rows1,
         acc_sh, sem0, sem1) = refs

    c = lax.axis_index("c")
    s = lax.axis_index("s")
    wid = c * NS + s

    pltpu.sync_copy(pk_hbm.at[wid], pk_v)

    def unpack(k, src_c, dst_c):
        for j in range(CH // 16):
            p = pk_v[k, pl.ds(j * 16, 16)]
            src_c[pl.ds(j * 16, 16)] = jnp.bitwise_and(p, 0xFFFF)
            dst_c[pl.ds(j * 16, 16)] = jnp.right_shift(p, 16)

    def scatter(rows, dst_c):
        pltpu.sync_copy(rows, acc_sh.at[dst_c], add=True)
        if compute_deg:
            pltpu.sync_copy(ones_v, deg_sh.at[dst_c], add=True)

    unpack(0, src_c0, dst_c0)
    unpack(1, src_c1, dst_c1)
    pltpu.async_copy(h_hbm.at[src_c0], rows0, sem0)
    pltpu.async_copy(h_hbm.at[src_c1], rows1, sem1)

    pltpu.sync_copy(z2_hbm, acc_sh.at[pl.ds(s * RPT, RPT)])
    if compute_deg:
        pltpu.sync_copy(z1_hbm, deg_sh.at[pl.ds(s * RPT, RPT)])
        pltpu.sync_copy(ones_hbm, ones_v)
    plsc.subcore_barrier()

    def step(k, src_c, dst_c, rows, sem, prefetch):
        pltpu.make_async_copy(h_hbm.at[src_c], rows, sem).wait()
        scatter(rows, dst_c)
        if prefetch:
            unpack(k + 2, src_c, dst_c)
            pltpu.async_copy(h_hbm.at[src_c], rows, sem)

    def body(i, carry):
        k = 2 * i
        step(k, src_c0, dst_c0, rows0, sem0, True)
        step(k + 1, src_c1, dst_c1, rows1, sem1, True)
        return carry

    if NCHUNK % 2 == 0:
        lax.fori_loop(0, NCHUNK // 2 - 1, body, 0)
        step(NCHUNK - 2, src_c0, dst_c0, rows0, sem0, False)
        step(NCHUNK - 1, src_c1, dst_c1, rows1, sem1, False)
    else:
        lax.fori_loop(0, (NCHUNK - 3) // 2, body, 0)
        step(NCHUNK - 3, src_c0, dst_c0, rows0, sem0, True)
        step(NCHUNK - 2, src_c1, dst_c1, rows1, sem1, False)
        step(NCHUNK - 1, src_c0, dst_c0, rows0, sem0, False)
    plsc.subcore_barrier()

    pltpu.sync_copy(acc_sh.at[pl.ds(s * RPT, RPT)],
                    acc_out.at[c, pl.ds(s * RPT, RPT)])
    if compute_deg:
        pltpu.sync_copy(deg_sh.at[pl.ds(s * RPT, RPT)],
                        deg_out.at[c, pl.ds(s * RPT, RPT)])


_sc_agg_deg = pl.kernel(
    functools.partial(_sc_agg_body, True),
    out_type=(
        jax.ShapeDtypeStruct((NC, NP, D), jnp.float32),
        jax.ShapeDtypeStruct((NC, NP), jnp.float32),
    ),
    mesh=_MESH,
    scratch_types=[
        pltpu.VMEM((NCHUNK, CH), jnp.int32),
        pltpu.VMEM((CH,), jnp.int32),
        pltpu.VMEM((CH,), jnp.int32),
        pltpu.VMEM((CH,), jnp.int32),
        pltpu.VMEM((CH,), jnp.int32),
        pltpu.VMEM((CH, D), jnp.float32),
        pltpu.VMEM((CH, D), jnp.float32),
        pltpu.VMEM((CH,), jnp.float32),
        pltpu.VMEM_SHARED((NP, D), jnp.float32),
        pltpu.VMEM_SHARED((NP,), jnp.float32),
        pltpu.SemaphoreType.DMA,
        pltpu.SemaphoreType.DMA,
    ],
    name="sc_agg_deg",
)

_sc_agg = pl.kernel(
    functools.partial(_sc_agg_body, False),
    out_type=jax.ShapeDtypeStruct((NC, NP, D), jnp.float32),
    mesh=_MESH,
    scratch_types=[
        pltpu.VMEM((NCHUNK, CH), jnp.int32),
        pltpu.VMEM((CH,), jnp.int32),
        pltpu.VMEM((CH,), jnp.int32),
        pltpu.VMEM((CH,), jnp.int32),
        pltpu.VMEM((CH,), jnp.int32),
        pltpu.VMEM((CH, D), jnp.float32),
        pltpu.VMEM((CH, D), jnp.float32),
        pltpu.VMEM_SHARED((NP, D), jnp.float32),
        pltpu.SemaphoreType.DMA,
        pltpu.SemaphoreType.DMA,
    ],
    name="sc_agg",
)

EB = E // 128


def _pack_body(ei_ref, out_ref):
    p = ei_ref[0:1, :] | (ei_ref[1:2, :] << 16)
    out_ref[...] = p.reshape(EB, 128)


_tc_pack = pl.pallas_call(
    _pack_body,
    grid=(1,),
    in_specs=[pl.BlockSpec((2, E), lambda i: (0, 0))],
    out_specs=pl.BlockSpec((EB, 128), lambda i: (0, 0)),
    out_shape=jax.ShapeDtypeStruct((EB, 128), jnp.int32),
)


BN = 1000


def _tc_lin_body(x_ref, w_ref, b_ref, out_ref):
    out_ref[...] = (jnp.dot(x_ref[...], w_ref[...],
                            preferred_element_type=jnp.float32)
                    + b_ref[...])


def _tc_comb1_body(lin_ref, a0_ref, a1_ref, deg_ref, wn_ref, h_out):
    n = (a0_ref[0] + a1_ref[0]) / deg_ref[...]
    h = lin_ref[...] + jnp.dot(n, wn_ref[...],
                               preferred_element_type=jnp.float32)
    h_out[...] = jnp.maximum(h, 0.0)


def _tc_comb2_body(lin_ref, a0_ref, a1_ref, deg_ref, wn_ref, out_ref):
    n = (a0_ref[0] + a1_ref[0]) / deg_ref[...]
    out_ref[...] = lin_ref[...] + jnp.dot(n, wn_ref[...],
                                          preferred_element_type=jnp.float32)


def _row_spec():
    return pl.BlockSpec((BN, D), lambda i: (i, 0))


def _acc_spec(c):
    return pl.BlockSpec((1, BN, D), lambda i, c=c: (c, i, 0))


def _col_spec():
    return pl.BlockSpec((BN, 1), lambda i: (i, 0))


def _full_spec():
    return pl.BlockSpec((D, D), lambda i: (0, 0))


def _bias_spec():
    return pl.BlockSpec((1, D), lambda i: (0, 0))


_tc_lin = pl.pallas_call(
    _tc_lin_body,
    grid=(N // BN,),
    in_specs=[_row_spec(), _full_spec(), _bias_spec()],
    out_specs=_row_spec(),
    out_shape=jax.ShapeDtypeStruct((N, D), jnp.float32),
)

_tc_comb1 = pl.pallas_call(
    _tc_comb1_body,
    grid=(N // BN,),
    in_specs=[_row_spec(), _acc_spec(0), _acc_spec(1), _col_spec(),
              _full_spec()],
    out_specs=_row_spec(),
    out_shape=jax.ShapeDtypeStruct((N, D), jnp.float32),
)

_tc_comb2 = pl.pallas_call(
    _tc_comb2_body,
    grid=(N // BN,),
    in_specs=[_row_spec(), _acc_spec(0), _acc_spec(1), _col_spec(),
              _full_spec()],
    out_specs=_row_spec(),
    out_shape=jax.ShapeDtypeStruct((N, D), jnp.float32),
)


def kernel(x, edge_index, W_self1, W_neigh1, b1, W_self2, W_neigh2, b2,
           W_fc, b_fc):
    ei = edge_index.astype(jnp.int32)
    ar = jnp.arange(EPAD, dtype=jnp.int32)
    pad = (ar % N) | ((N + ar % (NP - N)) << 16)
    pk = jnp.concatenate([_tc_pack(ei).reshape(E), pad])
    pk = pk.reshape(NW, NCHUNK, CH)
    z2 = jnp.zeros((RPT, D), jnp.float32)
    z1 = jnp.zeros((RPT,), jnp.float32)
    ones = jnp.ones((CH,), jnp.float32)

    lin1 = _tc_lin(x, W_self1.T, b1.reshape(1, D))
    acc1, degp = _sc_agg_deg(x, pk, z2, z1, ones)
    deg = jnp.maximum(degp[0, :N] + degp[1, :N], 1.0).reshape(N, 1)
    h1 = _tc_comb1(lin1, acc1, acc1, deg, W_neigh1.T)
    lin2 = _tc_lin(h1, (W_fc @ W_self2).T,
                   (b2 @ W_fc.T + b_fc).reshape(1, D))
    acc2 = _sc_agg(h1, pk, z2)
    out = _tc_comb2(lin2, acc2, acc2, deg, (W_fc @ W_neigh2).T)
    return out
